# Initial kernel scaffold; baseline (speedup 1.0000x reference)
import jax
import jax.numpy as jnp
from jax.experimental import pallas as pl


def kernel(x, edge_index, edge_attr, Wlin, blin, W1, b1, W2, b2, eps, gamma, beta):
    raise NotImplementedError("write your pallas kernel here")


# speedup vs baseline: 1.8296x; 1.8296x over previous
import functools

import jax
import jax.numpy as jnp
from jax import lax
from jax.experimental import pallas as pl
from jax.experimental.pallas import tpu as pltpu
from jax.experimental.pallas import tpu_sc as plsc


def _edge_linear_body(ea_ref, wl_ref, bl_ref, out_ref):
    out_ref[...] = (
        jnp.dot(ea_ref[...], wl_ref[...], preferred_element_type=jnp.float32)
        + bl_ref[...]
    )


def _edge_linear(edge_attr, Wlin, blin, H):
    E, D = edge_attr.shape
    BE = 1600
    nb = E // BE
    grid = (2, nb)
    return pl.pallas_call(
        _edge_linear_body,
        grid=grid,
        in_specs=[
            pl.BlockSpec((BE, D), lambda c, i: (i, 0)),
            pl.BlockSpec((D, H), lambda c, i: (0, c)),
            pl.BlockSpec((1, H), lambda c, i: (0, c)),
        ],
        out_specs=pl.BlockSpec((BE, H), lambda c, i: (c * nb + i, 0)),
        out_shape=jax.ShapeDtypeStruct((2 * E, H), jnp.float32),
    )(edge_attr, Wlin, blin.reshape(1, -1))


def _sc_aggregate(xh, eh, src, dst, N, E, H, K):
    NS = 16
    per_tile = E // NS
    n_chunks = per_tile // K
    G = H // 16
    B8 = (N // NS) // 8 * 8
    REM = N - B8 * NS
    ZR = 208
    n_zcopies = B8 // ZR
    assert B8 % ZR == 0 and REM % 8 == 0 and REM <= ZR

    mesh = plsc.VectorSubcoreMesh(core_axis_name="c", subcore_axis_name="s")

    @functools.partial(
        pl.kernel,
        out_type=jax.ShapeDtypeStruct((2 * N, H), jnp.float32),
        mesh=mesh,
        scratch_types=[
            pltpu.VMEM((K,), jnp.int32),
            pltpu.VMEM((K,), jnp.int32),
            pltpu.VMEM((K, H), jnp.float32),
            pltpu.VMEM((K, H), jnp.float32),
            pltpu.VMEM((ZR, H), jnp.float32),
            pltpu.VMEM_SHARED((N, H), jnp.float32),
            pltpu.SemaphoreType.DMA,
        ],
    )
    def body(xh_hbm, eh_hbm, src_hbm, dst_hbm, out_hbm,
             srcv, dstv, rows, ev, zbuf, acc, sem):
        cid = lax.axis_index("c")
        sid = lax.axis_index("s")

        def zero_row(j, c):
            for g in range(G):
                zbuf[j, pl.ds(g * 16, 16)] = jnp.zeros((16,), jnp.float32)
            return c

        lax.fori_loop(0, ZR, zero_row, 0)
        for i in range(n_zcopies):
            pltpu.sync_copy(
                zbuf, acc.at[pl.ds(pl.multiple_of(sid * B8 + i * ZR, 8), ZR)]
            )
        if REM:
            @pl.when(sid == NS - 1)
            def _():
                pltpu.sync_copy(zbuf.at[pl.ds(0, REM)],
                                acc.at[pl.ds(N - REM, REM)])
        plsc.subcore_barrier()

        def chunk(i, c):
            base = pl.multiple_of(sid * per_tile + i * K, 8)
            pltpu.sync_copy(src_hbm.at[pl.ds(base, K)], srcv)
            pltpu.sync_copy(dst_hbm.at[pl.ds(base, K)], dstv)
            off = cid * N
            for g in range(K // 16):
                srcv[pl.ds(g * 16, 16)] = srcv[pl.ds(g * 16, 16)] + off
            gath = pltpu.async_copy(xh_hbm.at[srcv], rows, sem)
            pltpu.sync_copy(
                eh_hbm.at[pl.ds(pl.multiple_of(cid * E + base, 8), K)], ev)
            gath.wait()

            def row(j, c2):
                for g in range(G):
                    sl = pl.ds(g * 16, 16)
                    ev[j, sl] = jnp.maximum(rows[j, sl] + ev[j, sl], 0.0)
                return c2

            lax.fori_loop(0, K, row, 0)
            pltpu.sync_copy(ev, acc.at[dstv], add=True)
            return c

        lax.fori_loop(0, n_chunks, chunk, 0)
        plsc.subcore_barrier()

        r0 = pl.multiple_of(sid * B8, 8)
        pltpu.sync_copy(
            acc.at[pl.ds(r0, B8)],
            out_hbm.at[pl.ds(pl.multiple_of(cid * N + sid * B8, 8), B8)],
        )
        if REM:
            @pl.when(sid == NS - 1)
            def _():
                pltpu.sync_copy(
                    acc.at[pl.ds(N - REM, REM)],
                    out_hbm.at[pl.ds(pl.multiple_of(cid * N + N - REM, 8),
                                     REM)],
                )

    return body(xh, eh, src, dst)


def _mlp_bn_body(x_ref, ah_ref, w1_ref, b1_ref, w2_ref, b2_ref,
                 eps_ref, gamma_ref, beta_ref, out_ref):
    n = x_ref.shape[0]
    x = x_ref[...]
    aggr = jnp.concatenate([ah_ref[:n, :], ah_ref[n:, :]], axis=1)
    h = (1.0 + eps_ref[0, 0]) * x + aggr
    h1 = jnp.maximum(
        jnp.dot(h, w1_ref[...], preferred_element_type=jnp.float32)
        + b1_ref[...], 0.0)
    h2 = (jnp.dot(h1, w2_ref[...], preferred_element_type=jnp.float32)
          + b2_ref[...])
    mean = jnp.mean(h2, axis=0, keepdims=True)
    var = jnp.mean((h2 - mean) ** 2, axis=0, keepdims=True)
    hn = (h2 - mean) * lax.rsqrt(var + 1e-5) * gamma_ref[...] + beta_ref[...]
    out_ref[...] = jnp.maximum(hn + x, 0.0)


def _mlp_bn(x, aggr2, W1, b1, W2, b2, eps, gamma, beta):
    N, F = x.shape
    return pl.pallas_call(
        _mlp_bn_body,
        out_shape=jax.ShapeDtypeStruct((N, F), jnp.float32),
    )(x, aggr2, W1, b1.reshape(1, -1), W2, b2.reshape(1, -1),
      eps.reshape(1, 1), gamma.reshape(1, -1), beta.reshape(1, -1))


def kernel(x, edge_index, edge_attr, Wlin, blin, W1, b1, W2, b2,
           eps, gamma, beta):
    N, F = x.shape
    E = edge_index.shape[1]
    H = F // 2
    src = edge_index[0]
    dst = edge_index[1]

    eh = _edge_linear(edge_attr, Wlin, blin, H)
    xh = jnp.concatenate([x[:, :H], x[:, H:]], axis=0)
    aggr2 = _sc_aggregate(xh, eh, src, dst, N, E, H, K=80)
    return _mlp_bn(x, aggr2, W1, b1, W2, b2, eps, gamma, beta)


# speedup vs baseline: 2.7307x; 1.4925x over previous
import functools

import jax
import jax.numpy as jnp
from jax import lax
from jax.experimental import pallas as pl
from jax.experimental.pallas import tpu as pltpu
from jax.experimental.pallas import tpu_sc as plsc


def _edge_linear_body(ea_ref, wl_ref, bl_ref, out_ref):
    out_ref[...] = (
        jnp.dot(ea_ref[...], wl_ref[...], preferred_element_type=jnp.float32)
        + bl_ref[...]
    )


def _edge_linear(edge_attr, Wlin, blin, H):
    E, D = edge_attr.shape
    BE = 1600
    nb = E // BE
    grid = (2, nb)
    return pl.pallas_call(
        _edge_linear_body,
        grid=grid,
        in_specs=[
            pl.BlockSpec((BE, D), lambda c, i: (i, 0)),
            pl.BlockSpec((D, H), lambda c, i: (0, c)),
            pl.BlockSpec((1, H), lambda c, i: (0, c)),
        ],
        out_specs=pl.BlockSpec((BE, H), lambda c, i: (c * nb + i, 0)),
        out_shape=jax.ShapeDtypeStruct((2 * E, H), jnp.float32),
    )(edge_attr, Wlin, blin.reshape(1, -1))


def _sc_aggregate(xh, eh, ei4, N, E, H, K, NB=3):
    NS = 16
    C = E // K
    cpt = C // NS
    NI = 3 * NB
    n_ring = (cpt // NI) * NI
    G = H // 16
    B8 = (N // NS) // 8 * 8
    REM = N - B8 * NS
    nz_full, nz_tail = B8 // K, B8 % K
    assert REM % 8 == 0 and REM <= K and nz_tail % 8 == 0

    mesh = plsc.VectorSubcoreMesh(core_axis_name="c", subcore_axis_name="s")

    scratch = (
        [pltpu.VMEM((2, K), jnp.int32) for _ in range(NI)]
        + [pltpu.VMEM((K, H), jnp.float32) for _ in range(3 * NB)]
        + [pltpu.VMEM_SHARED((N, H), jnp.float32)]
        + [pltpu.SemaphoreType.DMA for _ in range(NI + 3 * NB)]
    )

    @functools.partial(
        pl.kernel,
        out_type=jax.ShapeDtypeStruct((2 * N, H), jnp.float32),
        mesh=mesh,
        scratch_types=scratch,
    )
    def body(xh_hbm, eh_hbm, ei_hbm, out_hbm, *refs):
        idx = refs[0:NI]
        rows = refs[NI:NI + NB]
        ebuf = refs[NI + NB:NI + 2 * NB]
        msg = refs[NI + 2 * NB:NI + 3 * NB]
        acc = refs[NI + 3 * NB]
        si = refs[NI + 3 * NB + 1:2 * NI + 3 * NB + 1]
        sg = refs[2 * NI + 3 * NB + 1:2 * NI + 4 * NB + 1]
        se = refs[2 * NI + 4 * NB + 1:2 * NI + 5 * NB + 1]
        ssc = refs[2 * NI + 5 * NB + 1:2 * NI + 6 * NB + 1]

        cid = lax.axis_index("c")
        sid = lax.axis_index("s")
        base_chunk = sid * cpt

        def issue_idx(ib, t):
            pltpu.async_copy(ei_hbm.at[cid, base_chunk + t], idx[ib], si[ib])

        def wait_idx(ib):
            pltpu.make_async_copy(
                ei_hbm.at[cid, 0], idx[ib], si[ib]).wait()

        def issue_fetch(b, ib, t):
            pltpu.async_copy(xh_hbm.at[idx[ib].at[0]], rows[b], sg[b])
            erow = pl.multiple_of((cid * C + base_chunk + t) * K, 8)
            pltpu.async_copy(eh_hbm.at[pl.ds(erow, K)], ebuf[b], se[b])

        def wait_fetch(b):
            pltpu.make_async_copy(
                xh_hbm.at[pl.ds(0, K)], rows[b], sg[b]).wait()
            pltpu.make_async_copy(
                eh_hbm.at[pl.ds(0, K)], ebuf[b], se[b]).wait()

        def compute(b):
            def row(j, c):
                for g in range(G):
                    sl = pl.ds(g * 16, 16)
                    msg[b][j, sl] = jnp.maximum(
                        rows[b][j, sl] + ebuf[b][j, sl], 0.0)
                return c

            lax.fori_loop(0, K, row, 0)

        def issue_scatter(b, ib):
            pltpu.async_copy(msg[b], acc.at[idx[ib].at[1]], ssc[b], add=True)

        def wait_scatter(b):
            pltpu.make_async_copy(
                msg[b], acc.at[idx[0].at[1]], ssc[b]).wait()

        for j in range(min(2 * NB, cpt)):
            issue_idx(j, j)

        def zero_row(j, c):
            for g in range(G):
                msg[0][j, pl.ds(g * 16, 16)] = jnp.zeros((16,), jnp.float32)
            return c

        lax.fori_loop(0, K, zero_row, 0)
        for i in range(nz_full):
            pltpu.sync_copy(
                msg[0], acc.at[pl.ds(pl.multiple_of(sid * B8 + i * K, 8), K)])
        if nz_tail:
            pltpu.sync_copy(
                msg[0].at[pl.ds(0, nz_tail)],
                acc.at[pl.ds(pl.multiple_of(sid * B8 + nz_full * K, 8),
                             nz_tail)])
        if REM:
            @pl.when(sid == NS - 1)
            def _():
                pltpu.sync_copy(msg[0].at[pl.ds(0, REM)],
                                acc.at[pl.ds(N - REM, REM)])

        for b in range(NB):
            wait_idx(b)
            issue_fetch(b, b, b)
        plsc.subcore_barrier()

        def ring(q, c):
            t0 = q * NI
            for j in range(NI):
                t = t0 + j
                b = j % NB
                wait_fetch(b)

                @pl.when(t >= NB)
                def _():
                    wait_scatter(b)

                compute(b)
                issue_scatter(b, j)
                tn = t + 2 * NB

                @pl.when(tn < cpt)
                def _():
                    issue_idx((j + 2 * NB) % NI, tn)

                tf = t + NB

                @pl.when(tf < cpt)
                def _():
                    wait_idx((j + NB) % NI)
                    issue_fetch(b, (j + NB) % NI, tf)
            return c

        lax.fori_loop(0, n_ring // NI, ring, 0)
        for t in range(n_ring, cpt):
            j = t % NI
            b = j % NB
            wait_fetch(b)
            if t >= NB:
                wait_scatter(b)
            compute(b)
            issue_scatter(b, j)
            tn = t + 2 * NB
            if tn < cpt:
                issue_idx((j + 2 * NB) % NI, tn)
            tf = t + NB
            if tf < cpt:
                wait_idx((j + NB) % NI)
                issue_fetch(b, (j + NB) % NI, tf)
        for b in range(min(NB, cpt)):
            wait_scatter(b)
        plsc.subcore_barrier()

        r0 = pl.multiple_of(sid * B8, 8)
        pltpu.sync_copy(
            acc.at[pl.ds(r0, B8)],
            out_hbm.at[pl.ds(pl.multiple_of(cid * N + sid * B8, 8), B8)],
        )
        if REM:
            @pl.when(sid == NS - 1)
            def _():
                pltpu.sync_copy(
                    acc.at[pl.ds(N - REM, REM)],
                    out_hbm.at[pl.ds(pl.multiple_of(cid * N + N - REM, 8),
                                     REM)],
                )

    return body(xh, eh, ei4)


def _mlp_bn_body(x_ref, ah_ref, w1_ref, b1_ref, w2_ref, b2_ref,
                 eps_ref, gamma_ref, beta_ref, out_ref):
    n = x_ref.shape[0]
    x = x_ref[...]
    aggr = jnp.concatenate([ah_ref[:n, :], ah_ref[n:, :]], axis=1)
    h = (1.0 + eps_ref[0, 0]) * x + aggr
    h1 = jnp.maximum(
        jnp.dot(h, w1_ref[...], preferred_element_type=jnp.float32)
        + b1_ref[...], 0.0)
    h2 = (jnp.dot(h1, w2_ref[...], preferred_element_type=jnp.float32)
          + b2_ref[...])
    mean = jnp.mean(h2, axis=0, keepdims=True)
    var = jnp.mean((h2 - mean) ** 2, axis=0, keepdims=True)
    hn = (h2 - mean) * lax.rsqrt(var + 1e-5) * gamma_ref[...] + beta_ref[...]
    out_ref[...] = jnp.maximum(hn + x, 0.0)


def _mlp_bn(x, aggr2, W1, b1, W2, b2, eps, gamma, beta):
    N, F = x.shape
    return pl.pallas_call(
        _mlp_bn_body,
        out_shape=jax.ShapeDtypeStruct((N, F), jnp.float32),
    )(x, aggr2, W1, b1.reshape(1, -1), W2, b2.reshape(1, -1),
      eps.reshape(1, 1), gamma.reshape(1, -1), beta.reshape(1, -1))


def kernel(x, edge_index, edge_attr, Wlin, blin, W1, b1, W2, b2,
           eps, gamma, beta):
    N, F = x.shape
    E = edge_index.shape[1]
    H = F // 2
    K = 40
    src = edge_index[0]
    dst = edge_index[1]
    srcr = src.reshape(E // K, K)
    dstr = dst.reshape(E // K, K)
    ei4 = jnp.stack([jnp.stack([srcr, dstr], axis=1),
                     jnp.stack([srcr + N, dstr], axis=1)])

    eh = _edge_linear(edge_attr, Wlin, blin, H)
    xh = jnp.concatenate([x[:, :H], x[:, H:]], axis=0)
    aggr2 = _sc_aggregate(xh, eh, ei4, N, E, H, K)
    return _mlp_bn(x, aggr2, W1, b1, W2, b2, eps, gamma, beta)
